# baseline (speedup 1.0000x reference)
import functools

import jax
import jax.numpy as jnp
from jax import lax
from jax.experimental import pallas as pl
from jax.experimental.pallas import tpu as pltpu
from jax.experimental.pallas import tpu_sc as plsc

B = 4
N = 4096
M = 4096
K = 32
L = 16
ROWS_PER_W = (B * M) // 32
CHUNKS = M // ROWS_PER_W

_INF = float("inf")


def _sort16(k, v):
    return plsc.sort_key_val(k, v)


def _merge16(ak, ai, bk, bi):
    rk = lax.rev(bk, (0,))
    ri = lax.rev(bi, (0,))
    m = ak <= rk
    lok = jnp.where(m, ak, rk)
    loi = jnp.where(m, ai, ri)
    hik = jnp.where(m, rk, ak)
    hii = jnp.where(m, ri, ai)
    o0k, o0i = _sort16(lok, loi)
    o1k, o1i = _sort16(hik, hii)
    return o0k, o0i, o1k, o1i


def _low32(a0k, a0i, a1k, a1i, b0k, b0i, b1k, b1i):
    rb0k = lax.rev(b1k, (0,))
    rb0i = lax.rev(b1i, (0,))
    rb1k = lax.rev(b0k, (0,))
    rb1i = lax.rev(b0i, (0,))
    m0 = a0k <= rb0k
    c0k = jnp.where(m0, a0k, rb0k)
    c0i = jnp.where(m0, a0i, rb0i)
    m1 = a1k <= rb1k
    c1k = jnp.where(m1, a1k, rb1k)
    c1i = jnp.where(m1, a1i, rb1i)
    m = c0k <= c1k
    lk = jnp.where(m, c0k, c1k)
    li = jnp.where(m, c0i, c1i)
    hk = jnp.where(m, c1k, c0k)
    hi = jnp.where(m, c1i, c0i)
    o0k, o0i = _sort16(lk, li)
    o1k, o1i = _sort16(hk, hi)
    return o0k, o0i, o1k, o1i


def _fold16(sk, si, r0k, r0i, r1k, r1i):
    rsk = lax.rev(sk, (0,))
    rsi = lax.rev(si, (0,))
    m1 = r1k <= rsk
    c1k = jnp.where(m1, r1k, rsk)
    c1i = jnp.where(m1, r1i, rsi)
    m = r0k <= c1k
    lk = jnp.where(m, r0k, c1k)
    li = jnp.where(m, r0i, c1i)
    hk = jnp.where(m, c1k, r0k)
    hi = jnp.where(m, c1i, r0i)
    o0k, o0i = _sort16(lk, li)
    o1k, o1i = _sort16(hk, hi)
    return o0k, o0i, o1k, o1i


def _knn_body(x1_hbm, x2_hbm, out_hbm, cx, cy, cz, qx, qy, qz,
              dbuf, dbuf2, bufi, outv):
    info = plsc.get_sparse_core_info()
    nc = info.num_cores
    wid = lax.axis_index("s") * nc + lax.axis_index("c")
    b = wid // CHUNKS
    chunk = wid % CHUNKS
    row0 = chunk * ROWS_PER_W

    pltpu.sync_copy(x1_hbm.at[b * 3 + 0], cx)
    pltpu.sync_copy(x1_hbm.at[b * 3 + 1], cy)
    pltpu.sync_copy(x1_hbm.at[b * 3 + 2], cz)
    pltpu.sync_copy(x2_hbm.at[b * 3 + 0, pl.ds(row0, ROWS_PER_W)], qx)
    pltpu.sync_copy(x2_hbm.at[b * 3 + 1, pl.ds(row0, ROWS_PER_W)], qy)
    pltpu.sync_copy(x2_hbm.at[b * 3 + 2, pl.ds(row0, ROWS_PER_W)], qz)

    iota = lax.iota(jnp.int32, L)
    inf_vec = jnp.full((L,), _INF, jnp.float32)
    zero_vec = jnp.zeros((L,), jnp.int32)

    dbuf[pl.ds(N, L)] = jnp.full((L,), _INF, jnp.float32)
    dbuf2[pl.ds(N, L)] = jnp.full((L,), _INF, jnp.float32)

    def finish_row(db, mn0, mn1, m):
        t0 = jnp.max(jnp.maximum(mn0, mn1))
        t0v = jnp.full((L,), t0, jnp.float32)

        def pb(j, cnt):
            dv = db[pl.ds(j * L, L)]
            k = dv <= t0v
            pc = plsc.all_reduce_population_count(k)[0]
            plsc.store_compressed(bufi.at[pl.ds(cnt, L)], iota + j * L,
                                  mask=k)
            return cnt + pc

        cnt = plsc.parallel_loop(
            0, N // L, carry=jnp.int32(0), unroll=8)(pb)
        n_vec = jnp.full((L,), N, jnp.int32)
        bufi[pl.ds(cnt, L)] = n_vec
        bufi[pl.ds(cnt + L, L)] = n_vec
        bufi[pl.ds(cnt + 2 * L, L)] = n_vec

        def svreg(v):
            si = bufi[pl.ds(v * L, L)]
            sk = plsc.load_gather(db, [si])
            return _sort16(sk, si)

        s0k, s0i = svreg(0)
        s1k, s1i = svreg(1)
        ra = _merge16(s0k, s0i, s1k, s1i)
        rb = (inf_vec, zero_vec, inf_vec, zero_vec)

        def pc_body(u, carry):
            ra, rb = carry
            ak, ai = svreg(2 + 2 * u)
            bk, bi = svreg(3 + 2 * u)
            return _fold16(ak, ai, *ra), _fold16(bk, bi, *rb)

        nb2 = (cnt - 1) // (2 * L)
        ra, rb = lax.fori_loop(0, nb2, pc_body, (ra, rb))
        r0k, r0i, r1k, r1i = _low32(*ra, *rb)
        outv[pl.ds(m * K, L)] = r0i
        outv[pl.ds(m * K + L, L)] = r1i

    def row_body(m2, _):
        m = 2 * m2
        idxa = jnp.full((L,), m, jnp.int32)
        idxb = jnp.full((L,), m + 1, jnp.int32)
        qxa = plsc.load_gather(qx, [idxa])
        qya = plsc.load_gather(qy, [idxa])
        qza = plsc.load_gather(qz, [idxa])
        qxb = plsc.load_gather(qx, [idxb])
        qyb = plsc.load_gather(qy, [idxb])
        qzb = plsc.load_gather(qz, [idxb])

        def pa(j, carry):
            a0, a1, b0, b1 = carry
            base = j * 2 * L
            xv0 = cx[pl.ds(base, L)]
            yv0 = cy[pl.ds(base, L)]
            zv0 = cz[pl.ds(base, L)]
            xv1 = cx[pl.ds(base + L, L)]
            yv1 = cy[pl.ds(base + L, L)]
            zv1 = cz[pl.ds(base + L, L)]

            dxa0 = xv0 - qxa
            dya0 = yv0 - qya
            dza0 = zv0 - qza
            da0 = dxa0 * dxa0 + dya0 * dya0 + dza0 * dza0
            dxa1 = xv1 - qxa
            dya1 = yv1 - qya
            dza1 = zv1 - qza
            da1 = dxa1 * dxa1 + dya1 * dya1 + dza1 * dza1
            dxb0 = xv0 - qxb
            dyb0 = yv0 - qyb
            dzb0 = zv0 - qzb
            db0 = dxb0 * dxb0 + dyb0 * dyb0 + dzb0 * dzb0
            dxb1 = xv1 - qxb
            dyb1 = yv1 - qyb
            dzb1 = zv1 - qzb
            db1 = dxb1 * dxb1 + dyb1 * dyb1 + dzb1 * dzb1
            dbuf[pl.ds(base, L)] = da0
            dbuf[pl.ds(base + L, L)] = da1
            dbuf2[pl.ds(base, L)] = db0
            dbuf2[pl.ds(base + L, L)] = db1
            return (jnp.minimum(a0, da0), jnp.minimum(a1, da1),
                    jnp.minimum(b0, db0), jnp.minimum(b1, db1))

        a0, a1, b0, b1 = plsc.parallel_loop(
            0, N // (2 * L), carry=(inf_vec, inf_vec, inf_vec, inf_vec),
            unroll=4)(pa)
        finish_row(dbuf, a0, a1, m)
        finish_row(dbuf2, b0, b1, m + 1)
        return 0

    lax.fori_loop(0, ROWS_PER_W // 2, row_body, 0)
    pltpu.sync_copy(outv, out_hbm.at[pl.ds((b * M + row0) * K,
                                           ROWS_PER_W * K)])


@jax.jit
def _knn_sc(x1t, x2t):
    mesh = plsc.VectorSubcoreMesh(core_axis_name="c", subcore_axis_name="s")
    f = functools.partial(
        pl.kernel,
        out_type=jax.ShapeDtypeStruct((B * M * K,), jnp.int32),
        mesh=mesh,
        compiler_params=pltpu.CompilerParams(needs_layout_passes=False),
        scratch_types=[
            pltpu.VMEM((N,), jnp.float32),
            pltpu.VMEM((N,), jnp.float32),
            pltpu.VMEM((N,), jnp.float32),
            pltpu.VMEM((ROWS_PER_W,), jnp.float32),
            pltpu.VMEM((ROWS_PER_W,), jnp.float32),
            pltpu.VMEM((ROWS_PER_W,), jnp.float32),
            pltpu.VMEM((N + L,), jnp.float32),
            pltpu.VMEM((N + L,), jnp.float32),
            pltpu.VMEM((N + 4 * L,), jnp.int32),
            pltpu.VMEM((ROWS_PER_W * K,), jnp.int32),
        ],
    )(_knn_body)
    return f(x1t, x2t)


def kernel(xyz1, xyz2):
    x1t = xyz1.transpose(0, 2, 1).reshape(B * 3, N)
    x2t = xyz2.transpose(0, 2, 1).reshape(B * 3, M)
    out = _knn_sc(x1t, x2t)
    return out.reshape(B, M, K, 1)


# speedup vs baseline: 1.0223x; 1.0223x over previous
import functools

import jax
import jax.numpy as jnp
from jax import lax
from jax.experimental import pallas as pl
from jax.experimental.pallas import tpu as pltpu
from jax.experimental.pallas import tpu_sc as plsc

B = 4
N = 4096
M = 4096
K = 32
L = 16
ROWS_PER_W = (B * M) // 32
CHUNKS = M // ROWS_PER_W

_INF = float("inf")


def _sort16(k, v):
    return plsc.sort_key_val(k, v)


def _merge16(ak, ai, bk, bi):
    rk = lax.rev(bk, (0,))
    ri = lax.rev(bi, (0,))
    m = ak <= rk
    lok = jnp.where(m, ak, rk)
    loi = jnp.where(m, ai, ri)
    hik = jnp.where(m, rk, ak)
    hii = jnp.where(m, ri, ai)
    o0k, o0i = _sort16(lok, loi)
    o1k, o1i = _sort16(hik, hii)
    return o0k, o0i, o1k, o1i


def _low32(a0k, a0i, a1k, a1i, b0k, b0i, b1k, b1i):
    rb0k = lax.rev(b1k, (0,))
    rb0i = lax.rev(b1i, (0,))
    rb1k = lax.rev(b0k, (0,))
    rb1i = lax.rev(b0i, (0,))
    m0 = a0k <= rb0k
    c0k = jnp.where(m0, a0k, rb0k)
    c0i = jnp.where(m0, a0i, rb0i)
    m1 = a1k <= rb1k
    c1k = jnp.where(m1, a1k, rb1k)
    c1i = jnp.where(m1, a1i, rb1i)
    m = c0k <= c1k
    lk = jnp.where(m, c0k, c1k)
    li = jnp.where(m, c0i, c1i)
    hk = jnp.where(m, c1k, c0k)
    hi = jnp.where(m, c1i, c0i)
    o0k, o0i = _sort16(lk, li)
    o1k, o1i = _sort16(hk, hi)
    return o0k, o0i, o1k, o1i


def _fold16(sk, si, r0k, r0i, r1k, r1i):
    rsk = lax.rev(sk, (0,))
    rsi = lax.rev(si, (0,))
    m1 = r1k <= rsk
    c1k = jnp.where(m1, r1k, rsk)
    c1i = jnp.where(m1, r1i, rsi)
    m = r0k <= c1k
    lk = jnp.where(m, r0k, c1k)
    li = jnp.where(m, r0i, c1i)
    hk = jnp.where(m, c1k, r0k)
    hi = jnp.where(m, c1i, r0i)
    o0k, o0i = _sort16(lk, li)
    o1k, o1i = _sort16(hk, hi)
    return o0k, o0i, o1k, o1i


def _knn_body(x1_hbm, x2_hbm, out_hbm, cx, cy, cz, qx, qy, qz,
              dbuf, dbuf2, bufi, outv):
    info = plsc.get_sparse_core_info()
    nc = info.num_cores
    wid = lax.axis_index("s") * nc + lax.axis_index("c")
    b = wid // CHUNKS
    chunk = wid % CHUNKS
    row0 = chunk * ROWS_PER_W

    pltpu.sync_copy(x1_hbm.at[b * 3 + 0], cx)
    pltpu.sync_copy(x1_hbm.at[b * 3 + 1], cy)
    pltpu.sync_copy(x1_hbm.at[b * 3 + 2], cz)
    pltpu.sync_copy(x2_hbm.at[b * 3 + 0, pl.ds(row0, ROWS_PER_W)], qx)
    pltpu.sync_copy(x2_hbm.at[b * 3 + 1, pl.ds(row0, ROWS_PER_W)], qy)
    pltpu.sync_copy(x2_hbm.at[b * 3 + 2, pl.ds(row0, ROWS_PER_W)], qz)

    iota = lax.iota(jnp.int32, L)
    inf_vec = jnp.full((L,), _INF, jnp.float32)
    zero_vec = jnp.zeros((L,), jnp.int32)

    dbuf[pl.ds(N, L)] = jnp.full((L,), _INF, jnp.float32)
    dbuf2[pl.ds(N, L)] = jnp.full((L,), _INF, jnp.float32)

    def finish_row(db, mn0, mn1, m):
        t0 = jnp.max(jnp.maximum(mn0, mn1))
        t0v = jnp.full((L,), t0, jnp.float32)

        def pb(j, cnt):
            dv = db[pl.ds(j * L, L)]
            k = dv <= t0v
            pc = plsc.all_reduce_population_count(k)[0]
            plsc.store_compressed(bufi.at[pl.ds(cnt, L)], iota + j * L,
                                  mask=k)
            return cnt + pc

        cnt = plsc.parallel_loop(
            0, N // L, carry=jnp.int32(0), unroll=8)(pb)
        n_vec = jnp.full((L,), N, jnp.int32)
        bufi[pl.ds(cnt, L)] = n_vec
        bufi[pl.ds(cnt + L, L)] = n_vec
        bufi[pl.ds(cnt + 2 * L, L)] = n_vec

        def svreg(v):
            si = bufi[pl.ds(v * L, L)]
            sk = plsc.load_gather(db, [si])
            return _sort16(sk, si)

        s0k, s0i = svreg(0)
        s1k, s1i = svreg(1)
        ra = _merge16(s0k, s0i, s1k, s1i)
        rb = (inf_vec, zero_vec, inf_vec, zero_vec)

        def pc_body(u, carry):
            ra, rb = carry
            ak, ai = svreg(2 + 2 * u)
            bk, bi = svreg(3 + 2 * u)
            return _fold16(ak, ai, *ra), _fold16(bk, bi, *rb)

        nb2 = (cnt - 1) // (2 * L)
        ra, rb = lax.fori_loop(0, nb2, pc_body, (ra, rb))
        r0k, r0i, r1k, r1i = _low32(*ra, *rb)
        outv[pl.ds(m * K, L)] = r0i
        outv[pl.ds(m * K + L, L)] = r1i

    def row_body(m2, _):
        m = 2 * m2
        idxa = jnp.full((L,), m, jnp.int32)
        idxb = jnp.full((L,), m + 1, jnp.int32)
        qxa = plsc.load_gather(qx, [idxa])
        qya = plsc.load_gather(qy, [idxa])
        qza = plsc.load_gather(qz, [idxa])
        qxb = plsc.load_gather(qx, [idxb])
        qyb = plsc.load_gather(qy, [idxb])
        qzb = plsc.load_gather(qz, [idxb])

        def pa(j, carry):
            a0, a1, b0, b1 = carry
            base = j * 2 * L
            xv0 = cx[pl.ds(base, L)]
            yv0 = cy[pl.ds(base, L)]
            zv0 = cz[pl.ds(base, L)]
            xv1 = cx[pl.ds(base + L, L)]
            yv1 = cy[pl.ds(base + L, L)]
            zv1 = cz[pl.ds(base + L, L)]

            dxa0 = xv0 - qxa
            dya0 = yv0 - qya
            dza0 = zv0 - qza
            da0 = dxa0 * dxa0 + dya0 * dya0 + dza0 * dza0
            dxa1 = xv1 - qxa
            dya1 = yv1 - qya
            dza1 = zv1 - qza
            da1 = dxa1 * dxa1 + dya1 * dya1 + dza1 * dza1
            dxb0 = xv0 - qxb
            dyb0 = yv0 - qyb
            dzb0 = zv0 - qzb
            db0 = dxb0 * dxb0 + dyb0 * dyb0 + dzb0 * dzb0
            dxb1 = xv1 - qxb
            dyb1 = yv1 - qyb
            dzb1 = zv1 - qzb
            db1 = dxb1 * dxb1 + dyb1 * dyb1 + dzb1 * dzb1
            dbuf[pl.ds(base, L)] = da0
            dbuf[pl.ds(base + L, L)] = da1
            dbuf2[pl.ds(base, L)] = db0
            dbuf2[pl.ds(base + L, L)] = db1
            return (jnp.minimum(a0, da0), jnp.minimum(a1, da1),
                    jnp.minimum(b0, db0), jnp.minimum(b1, db1))

        a0, a1, b0, b1 = plsc.parallel_loop(
            0, N // (2 * L), carry=(inf_vec, inf_vec, inf_vec, inf_vec),
            unroll=2)(pa)
        finish_row(dbuf, a0, a1, m)
        finish_row(dbuf2, b0, b1, m + 1)
        return 0

    lax.fori_loop(0, ROWS_PER_W // 2, row_body, 0)
    pltpu.sync_copy(outv, out_hbm.at[pl.ds((b * M + row0) * K,
                                           ROWS_PER_W * K)])


@jax.jit
def _knn_sc(x1t, x2t):
    mesh = plsc.VectorSubcoreMesh(core_axis_name="c", subcore_axis_name="s")
    f = functools.partial(
        pl.kernel,
        out_type=jax.ShapeDtypeStruct((B * M * K,), jnp.int32),
        mesh=mesh,
        compiler_params=pltpu.CompilerParams(needs_layout_passes=False),
        scratch_types=[
            pltpu.VMEM((N,), jnp.float32),
            pltpu.VMEM((N,), jnp.float32),
            pltpu.VMEM((N,), jnp.float32),
            pltpu.VMEM((ROWS_PER_W,), jnp.float32),
            pltpu.VMEM((ROWS_PER_W,), jnp.float32),
            pltpu.VMEM((ROWS_PER_W,), jnp.float32),
            pltpu.VMEM((N + L,), jnp.float32),
            pltpu.VMEM((N + L,), jnp.float32),
            pltpu.VMEM((N + 4 * L,), jnp.int32),
            pltpu.VMEM((ROWS_PER_W * K,), jnp.int32),
        ],
    )(_knn_body)
    return f(x1t, x2t)


def kernel(xyz1, xyz2):
    x1t = xyz1.transpose(0, 2, 1).reshape(B * 3, N)
    x2t = xyz2.transpose(0, 2, 1).reshape(B * 3, M)
    out = _knn_sc(x1t, x2t)
    return out.reshape(B, M, K, 1)


# speedup vs baseline: 1.0653x; 1.0420x over previous
import functools

import jax
import jax.numpy as jnp
from jax import lax
from jax.experimental import pallas as pl
from jax.experimental.pallas import tpu as pltpu
from jax.experimental.pallas import tpu_sc as plsc

B = 4
N = 4096
M = 4096
K = 32
L = 16
ROWS_PER_W = (B * M) // 32
CHUNKS = M // ROWS_PER_W

_INF = float("inf")


def _sort16(k, v):
    return plsc.sort_key_val(k, v)


def _merge16(ak, ai, bk, bi):
    rk = lax.rev(bk, (0,))
    ri = lax.rev(bi, (0,))
    m = ak <= rk
    lok = jnp.where(m, ak, rk)
    loi = jnp.where(m, ai, ri)
    hik = jnp.where(m, rk, ak)
    hii = jnp.where(m, ri, ai)
    o0k, o0i = _sort16(lok, loi)
    o1k, o1i = _sort16(hik, hii)
    return o0k, o0i, o1k, o1i


def _low32(a0k, a0i, a1k, a1i, b0k, b0i, b1k, b1i):
    rb0k = lax.rev(b1k, (0,))
    rb0i = lax.rev(b1i, (0,))
    rb1k = lax.rev(b0k, (0,))
    rb1i = lax.rev(b0i, (0,))
    m0 = a0k <= rb0k
    c0k = jnp.where(m0, a0k, rb0k)
    c0i = jnp.where(m0, a0i, rb0i)
    m1 = a1k <= rb1k
    c1k = jnp.where(m1, a1k, rb1k)
    c1i = jnp.where(m1, a1i, rb1i)
    m = c0k <= c1k
    lk = jnp.where(m, c0k, c1k)
    li = jnp.where(m, c0i, c1i)
    hk = jnp.where(m, c1k, c0k)
    hi = jnp.where(m, c1i, c0i)
    o0k, o0i = _sort16(lk, li)
    o1k, o1i = _sort16(hk, hi)
    return o0k, o0i, o1k, o1i


def _fold16(sk, si, r0k, r0i, r1k, r1i):
    rsk = lax.rev(sk, (0,))
    rsi = lax.rev(si, (0,))
    m1 = r1k <= rsk
    c1k = jnp.where(m1, r1k, rsk)
    c1i = jnp.where(m1, r1i, rsi)
    m = r0k <= c1k
    lk = jnp.where(m, r0k, c1k)
    li = jnp.where(m, r0i, c1i)
    hk = jnp.where(m, c1k, r0k)
    hi = jnp.where(m, c1i, r0i)
    o0k, o0i = _sort16(lk, li)
    o1k, o1i = _sort16(hk, hi)
    return o0k, o0i, o1k, o1i


def _knn_body(x1_hbm, x2_hbm, out_hbm, cx, cy, cz, qx, qy, qz,
              dbuf, dbuf2, bufi, bufi2, outv):
    info = plsc.get_sparse_core_info()
    nc = info.num_cores
    wid = lax.axis_index("s") * nc + lax.axis_index("c")
    b = wid // CHUNKS
    chunk = wid % CHUNKS
    row0 = chunk * ROWS_PER_W

    pltpu.sync_copy(x1_hbm.at[b * 3 + 0], cx)
    pltpu.sync_copy(x1_hbm.at[b * 3 + 1], cy)
    pltpu.sync_copy(x1_hbm.at[b * 3 + 2], cz)
    pltpu.sync_copy(x2_hbm.at[b * 3 + 0, pl.ds(row0, ROWS_PER_W)], qx)
    pltpu.sync_copy(x2_hbm.at[b * 3 + 1, pl.ds(row0, ROWS_PER_W)], qy)
    pltpu.sync_copy(x2_hbm.at[b * 3 + 2, pl.ds(row0, ROWS_PER_W)], qz)

    iota = lax.iota(jnp.int32, L)
    inf_vec = jnp.full((L,), _INF, jnp.float32)
    zero_vec = jnp.zeros((L,), jnp.int32)

    dbuf[pl.ds(N, L)] = jnp.full((L,), _INF, jnp.float32)
    dbuf2[pl.ds(N, L)] = jnp.full((L,), _INF, jnp.float32)

    def finish_pair(mna0, mna1, mnb0, mnb1, m):
        t0a = jnp.max(jnp.maximum(mna0, mna1))
        t0b = jnp.max(jnp.maximum(mnb0, mnb1))
        t0av = jnp.full((L,), t0a, jnp.float32)
        t0bv = jnp.full((L,), t0b, jnp.float32)

        def pb(j, carry):
            ca, cb = carry
            iv = iota + j * L
            dva = dbuf[pl.ds(j * L, L)]
            ka = dva <= t0av
            pca = plsc.all_reduce_population_count(ka)[0]
            plsc.store_compressed(bufi.at[pl.ds(ca, L)], iv, mask=ka)
            dvb = dbuf2[pl.ds(j * L, L)]
            kb = dvb <= t0bv
            pcb = plsc.all_reduce_population_count(kb)[0]
            plsc.store_compressed(bufi2.at[pl.ds(cb, L)], iv, mask=kb)
            return ca + pca, cb + pcb

        cnta, cntb = plsc.parallel_loop(
            0, N // L, carry=(jnp.int32(0), jnp.int32(0)), unroll=4)(pb)
        n_vec = jnp.full((L,), N, jnp.int32)
        bufi[pl.ds(cnta, L)] = n_vec
        bufi[pl.ds(cnta + L, L)] = n_vec
        bufi[pl.ds(cnta + 2 * L, L)] = n_vec
        bufi2[pl.ds(cntb, L)] = n_vec
        bufi2[pl.ds(cntb + L, L)] = n_vec
        bufi2[pl.ds(cntb + 2 * L, L)] = n_vec

        def svreg(buf, db, v):
            si = buf[pl.ds(v * L, L)]
            sk = plsc.load_gather(db, [si])
            return _sort16(sk, si)

        nva = (cnta + (L - 1)) // L
        nvb = (cntb + (L - 1)) // L
        ra1 = _merge16(*svreg(bufi, dbuf, 0), *svreg(bufi, dbuf, 1))
        ra2 = (inf_vec, zero_vec, inf_vec, zero_vec)
        rb1 = _merge16(*svreg(bufi2, dbuf2, 0), *svreg(bufi2, dbuf2, 1))
        rb2 = (inf_vec, zero_vec, inf_vec, zero_vec)

        def pc_body(u, carry):
            ra1, ra2, rb1, rb2 = carry
            va0 = jnp.minimum(2 + 2 * u, nva)
            va1 = jnp.minimum(3 + 2 * u, nva)
            vb0 = jnp.minimum(2 + 2 * u, nvb)
            vb1 = jnp.minimum(3 + 2 * u, nvb)
            ra1 = _fold16(*svreg(bufi, dbuf, va0), *ra1)
            ra2 = _fold16(*svreg(bufi, dbuf, va1), *ra2)
            rb1 = _fold16(*svreg(bufi2, dbuf2, vb0), *rb1)
            rb2 = _fold16(*svreg(bufi2, dbuf2, vb1), *rb2)
            return ra1, ra2, rb1, rb2

        umax = jnp.maximum((cnta - 1) // (2 * L), (cntb - 1) // (2 * L))
        ra1, ra2, rb1, rb2 = lax.fori_loop(
            0, umax, pc_body, (ra1, ra2, rb1, rb2))
        r0k, r0i, r1k, r1i = _low32(*ra1, *ra2)
        outv[pl.ds(m * K, L)] = r0i
        outv[pl.ds(m * K + L, L)] = r1i
        s0k, s0i, s1k, s1i = _low32(*rb1, *rb2)
        outv[pl.ds((m + 1) * K, L)] = s0i
        outv[pl.ds((m + 1) * K + L, L)] = s1i

    def row_body(m2, _):
        m = 2 * m2
        idxa = jnp.full((L,), m, jnp.int32)
        idxb = jnp.full((L,), m + 1, jnp.int32)
        qxa = plsc.load_gather(qx, [idxa])
        qya = plsc.load_gather(qy, [idxa])
        qza = plsc.load_gather(qz, [idxa])
        qxb = plsc.load_gather(qx, [idxb])
        qyb = plsc.load_gather(qy, [idxb])
        qzb = plsc.load_gather(qz, [idxb])

        def pa(j, carry):
            a0, a1, b0, b1 = carry
            base = j * 2 * L
            xv0 = cx[pl.ds(base, L)]
            yv0 = cy[pl.ds(base, L)]
            zv0 = cz[pl.ds(base, L)]
            xv1 = cx[pl.ds(base + L, L)]
            yv1 = cy[pl.ds(base + L, L)]
            zv1 = cz[pl.ds(base + L, L)]

            dxa0 = xv0 - qxa
            dya0 = yv0 - qya
            dza0 = zv0 - qza
            da0 = dxa0 * dxa0 + dya0 * dya0 + dza0 * dza0
            dxa1 = xv1 - qxa
            dya1 = yv1 - qya
            dza1 = zv1 - qza
            da1 = dxa1 * dxa1 + dya1 * dya1 + dza1 * dza1
            dxb0 = xv0 - qxb
            dyb0 = yv0 - qyb
            dzb0 = zv0 - qzb
            db0 = dxb0 * dxb0 + dyb0 * dyb0 + dzb0 * dzb0
            dxb1 = xv1 - qxb
            dyb1 = yv1 - qyb
            dzb1 = zv1 - qzb
            db1 = dxb1 * dxb1 + dyb1 * dyb1 + dzb1 * dzb1
            dbuf[pl.ds(base, L)] = da0
            dbuf[pl.ds(base + L, L)] = da1
            dbuf2[pl.ds(base, L)] = db0
            dbuf2[pl.ds(base + L, L)] = db1
            return (jnp.minimum(a0, da0), jnp.minimum(a1, da1),
                    jnp.minimum(b0, db0), jnp.minimum(b1, db1))

        a0, a1, b0, b1 = plsc.parallel_loop(
            0, N // (2 * L), carry=(inf_vec, inf_vec, inf_vec, inf_vec),
            unroll=2)(pa)
        finish_pair(a0, a1, b0, b1, m)
        return 0

    lax.fori_loop(0, ROWS_PER_W // 2, row_body, 0)
    pltpu.sync_copy(outv, out_hbm.at[pl.ds((b * M + row0) * K,
                                           ROWS_PER_W * K)])


@jax.jit
def _knn_sc(x1t, x2t):
    mesh = plsc.VectorSubcoreMesh(core_axis_name="c", subcore_axis_name="s")
    f = functools.partial(
        pl.kernel,
        out_type=jax.ShapeDtypeStruct((B * M * K,), jnp.int32),
        mesh=mesh,
        compiler_params=pltpu.CompilerParams(needs_layout_passes=False),
        scratch_types=[
            pltpu.VMEM((N,), jnp.float32),
            pltpu.VMEM((N,), jnp.float32),
            pltpu.VMEM((N,), jnp.float32),
            pltpu.VMEM((ROWS_PER_W,), jnp.float32),
            pltpu.VMEM((ROWS_PER_W,), jnp.float32),
            pltpu.VMEM((ROWS_PER_W,), jnp.float32),
            pltpu.VMEM((N + L,), jnp.float32),
            pltpu.VMEM((N + L,), jnp.float32),
            pltpu.VMEM((N + 4 * L,), jnp.int32),
            pltpu.VMEM((N + 4 * L,), jnp.int32),
            pltpu.VMEM((ROWS_PER_W * K,), jnp.int32),
        ],
    )(_knn_body)
    return f(x1t, x2t)


def kernel(xyz1, xyz2):
    x1t = xyz1.transpose(0, 2, 1).reshape(B * 3, N)
    x2t = xyz2.transpose(0, 2, 1).reshape(B * 3, M)
    out = _knn_sc(x1t, x2t)
    return out.reshape(B, M, K, 1)


# speedup vs baseline: 1.1537x; 1.0831x over previous
import functools

import jax
import jax.numpy as jnp
from jax import lax
from jax.experimental import pallas as pl
from jax.experimental.pallas import tpu as pltpu
from jax.experimental.pallas import tpu_sc as plsc

B = 4
N = 4096
M = 4096
K = 32
L = 16
ROWS_PER_W = (B * M) // 32
CHUNKS = M // ROWS_PER_W

_INF = float("inf")


def _sort16(k, v):
    return plsc.sort_key_val(k, v)


def _merge16(ak, ai, bk, bi):
    rk = lax.rev(bk, (0,))
    ri = lax.rev(bi, (0,))
    m = ak <= rk
    lok = jnp.where(m, ak, rk)
    loi = jnp.where(m, ai, ri)
    hik = jnp.where(m, rk, ak)
    hii = jnp.where(m, ri, ai)
    o0k, o0i = _sort16(lok, loi)
    o1k, o1i = _sort16(hik, hii)
    return o0k, o0i, o1k, o1i


def _low32(a0k, a0i, a1k, a1i, b0k, b0i, b1k, b1i):
    rb0k = lax.rev(b1k, (0,))
    rb0i = lax.rev(b1i, (0,))
    rb1k = lax.rev(b0k, (0,))
    rb1i = lax.rev(b0i, (0,))
    m0 = a0k <= rb0k
    c0k = jnp.where(m0, a0k, rb0k)
    c0i = jnp.where(m0, a0i, rb0i)
    m1 = a1k <= rb1k
    c1k = jnp.where(m1, a1k, rb1k)
    c1i = jnp.where(m1, a1i, rb1i)
    m = c0k <= c1k
    lk = jnp.where(m, c0k, c1k)
    li = jnp.where(m, c0i, c1i)
    hk = jnp.where(m, c1k, c0k)
    hi = jnp.where(m, c1i, c0i)
    o0k, o0i = _sort16(lk, li)
    o1k, o1i = _sort16(hk, hi)
    return o0k, o0i, o1k, o1i


def _fold16(sk, si, r0k, r0i, r1k, r1i):
    rsk = lax.rev(sk, (0,))
    rsi = lax.rev(si, (0,))
    m1 = r1k <= rsk
    c1k = jnp.where(m1, r1k, rsk)
    c1i = jnp.where(m1, r1i, rsi)
    m = r0k <= c1k
    lk = jnp.where(m, r0k, c1k)
    li = jnp.where(m, r0i, c1i)
    hk = jnp.where(m, c1k, r0k)
    hi = jnp.where(m, c1i, r0i)
    o0k, o0i = _sort16(lk, li)
    o1k, o1i = _sort16(hk, hi)
    return o0k, o0i, o1k, o1i


def _knn_body(x1_hbm, x2_hbm, out_hbm, cx, cy, cz, qx, qy, qz,
              cs, dbuf, dbuf2, bufi, bufi2, outv):
    info = plsc.get_sparse_core_info()
    nc = info.num_cores
    wid = lax.axis_index("s") * nc + lax.axis_index("c")
    b = wid // CHUNKS
    chunk = wid % CHUNKS
    row0 = chunk * ROWS_PER_W

    pltpu.sync_copy(x1_hbm.at[b * 3 + 0], cx.at[pl.ds(0, N)])
    pltpu.sync_copy(x1_hbm.at[b * 3 + 1], cy.at[pl.ds(0, N)])
    pltpu.sync_copy(x1_hbm.at[b * 3 + 2], cz.at[pl.ds(0, N)])
    pltpu.sync_copy(x2_hbm.at[b * 3 + 0, pl.ds(row0, ROWS_PER_W)], qx)
    pltpu.sync_copy(x2_hbm.at[b * 3 + 1, pl.ds(row0, ROWS_PER_W)], qy)
    pltpu.sync_copy(x2_hbm.at[b * 3 + 2, pl.ds(row0, ROWS_PER_W)], qz)

    iota = lax.iota(jnp.int32, L)
    inf_vec = jnp.full((L,), _INF, jnp.float32)
    zero_vec = jnp.zeros((L,), jnp.int32)

    dbuf[pl.ds(N, L)] = jnp.full((L,), _INF, jnp.float32)
    dbuf2[pl.ds(N, L)] = jnp.full((L,), _INF, jnp.float32)
    cx[pl.ds(N, L)] = jnp.full((L,), _INF, jnp.float32)
    cy[pl.ds(N, L)] = jnp.full((L,), _INF, jnp.float32)
    cz[pl.ds(N, L)] = jnp.full((L,), _INF, jnp.float32)

    def ps(j, smx):
        xv = cx[pl.ds(j * L, L)]
        yv = cy[pl.ds(j * L, L)]
        zv = cz[pl.ds(j * L, L)]
        sv = xv * xv + yv * yv + zv * zv
        cs[pl.ds(j * L, L)] = sv
        return jnp.maximum(smx, sv)

    smxv = plsc.parallel_loop(
        0, N // L, carry=jnp.full((L,), 0.0, jnp.float32), unroll=4)(ps)
    smax_s = jnp.max(smxv)

    def finish_pair(mna0, mna1, mnb0, mnb1, m, qa, qb):
        qxa, qya, qza = qa
        qxb, qyb, qzb = qb
        smaxv = jnp.full((L,), smax_s, jnp.float32)
        q2a = qxa * qxa + qya * qya + qza * qza
        q2b = qxb * qxb + qyb * qyb + qzb * qzb
        eps = jnp.full((L,), 1e-5, jnp.float32)
        one = jnp.full((L,), 1.0, jnp.float32)
        t0a = jnp.max(jnp.maximum(mna0, mna1))
        t0b = jnp.max(jnp.maximum(mnb0, mnb1))
        t0av = jnp.full((L,), t0a, jnp.float32) + eps * (smaxv + q2a + one)
        t0bv = jnp.full((L,), t0b, jnp.float32) + eps * (smaxv + q2b + one)

        def pb(j, carry):
            ca, cb = carry
            iv = iota + j * L
            dva = dbuf[pl.ds(j * L, L)]
            ka = dva <= t0av
            pca = plsc.all_reduce_population_count(ka)[0]
            plsc.store_compressed(bufi.at[pl.ds(ca, L)], iv, mask=ka)
            dvb = dbuf2[pl.ds(j * L, L)]
            kb = dvb <= t0bv
            pcb = plsc.all_reduce_population_count(kb)[0]
            plsc.store_compressed(bufi2.at[pl.ds(cb, L)], iv, mask=kb)
            return ca + pca, cb + pcb

        cnta, cntb = plsc.parallel_loop(
            0, N // L, carry=(jnp.int32(0), jnp.int32(0)), unroll=4)(pb)
        n_vec = jnp.full((L,), N, jnp.int32)
        bufi[pl.ds(cnta, L)] = n_vec
        bufi[pl.ds(cnta + L, L)] = n_vec
        bufi[pl.ds(cnta + 2 * L, L)] = n_vec
        bufi2[pl.ds(cntb, L)] = n_vec
        bufi2[pl.ds(cntb + L, L)] = n_vec
        bufi2[pl.ds(cntb + 2 * L, L)] = n_vec

        def svreg(buf, q3, v):
            qxv, qyv, qzv = q3
            si = buf[pl.ds(v * L, L)]
            gx = plsc.load_gather(cx, [si]) - qxv
            gy = plsc.load_gather(cy, [si]) - qyv
            gz = plsc.load_gather(cz, [si]) - qzv
            sk = gx * gx + gy * gy + gz * gz
            return _sort16(sk, si)

        nva = (cnta + (L - 1)) // L
        nvb = (cntb + (L - 1)) // L
        ra1 = _merge16(*svreg(bufi, qa, 0), *svreg(bufi, qa, 1))
        ra2 = (inf_vec, zero_vec, inf_vec, zero_vec)
        rb1 = _merge16(*svreg(bufi2, qb, 0), *svreg(bufi2, qb, 1))
        rb2 = (inf_vec, zero_vec, inf_vec, zero_vec)

        def pc_body(u, carry):
            ra1, ra2, rb1, rb2 = carry
            va0 = jnp.minimum(2 + 2 * u, nva)
            va1 = jnp.minimum(3 + 2 * u, nva)
            vb0 = jnp.minimum(2 + 2 * u, nvb)
            vb1 = jnp.minimum(3 + 2 * u, nvb)
            ra1 = _fold16(*svreg(bufi, qa, va0), *ra1)
            ra2 = _fold16(*svreg(bufi, qa, va1), *ra2)
            rb1 = _fold16(*svreg(bufi2, qb, vb0), *rb1)
            rb2 = _fold16(*svreg(bufi2, qb, vb1), *rb2)
            return ra1, ra2, rb1, rb2

        umax = jnp.maximum((cnta - 1) // (2 * L), (cntb - 1) // (2 * L))
        ra1, ra2, rb1, rb2 = lax.fori_loop(
            0, umax, pc_body, (ra1, ra2, rb1, rb2))
        r0k, r0i, r1k, r1i = _low32(*ra1, *ra2)
        outv[pl.ds(m * K, L)] = r0i
        outv[pl.ds(m * K + L, L)] = r1i
        s0k, s0i, s1k, s1i = _low32(*rb1, *rb2)
        outv[pl.ds((m + 1) * K, L)] = s0i
        outv[pl.ds((m + 1) * K + L, L)] = s1i

    def row_body(m2, _):
        m = 2 * m2
        idxa = jnp.full((L,), m, jnp.int32)
        idxb = jnp.full((L,), m + 1, jnp.int32)
        qxa = plsc.load_gather(qx, [idxa])
        qya = plsc.load_gather(qy, [idxa])
        qza = plsc.load_gather(qz, [idxa])
        qxb = plsc.load_gather(qx, [idxb])
        qyb = plsc.load_gather(qy, [idxb])
        qzb = plsc.load_gather(qz, [idxb])
        mxa = jnp.float32(-2.0) * qxa
        mya = jnp.float32(-2.0) * qya
        mza = jnp.float32(-2.0) * qza
        mxb = jnp.float32(-2.0) * qxb
        myb = jnp.float32(-2.0) * qyb
        mzb = jnp.float32(-2.0) * qzb

        def pa(j, carry):
            a0, a1, b0, b1 = carry
            base = j * 2 * L
            xv0 = cx[pl.ds(base, L)]
            yv0 = cy[pl.ds(base, L)]
            zv0 = cz[pl.ds(base, L)]
            sv0 = cs[pl.ds(base, L)]
            xv1 = cx[pl.ds(base + L, L)]
            yv1 = cy[pl.ds(base + L, L)]
            zv1 = cz[pl.ds(base + L, L)]
            sv1 = cs[pl.ds(base + L, L)]

            da0 = sv0 + xv0 * mxa + yv0 * mya + zv0 * mza
            da1 = sv1 + xv1 * mxa + yv1 * mya + zv1 * mza
            db0 = sv0 + xv0 * mxb + yv0 * myb + zv0 * mzb
            db1 = sv1 + xv1 * mxb + yv1 * myb + zv1 * mzb
            dbuf[pl.ds(base, L)] = da0
            dbuf[pl.ds(base + L, L)] = da1
            dbuf2[pl.ds(base, L)] = db0
            dbuf2[pl.ds(base + L, L)] = db1
            return (jnp.minimum(a0, da0), jnp.minimum(a1, da1),
                    jnp.minimum(b0, db0), jnp.minimum(b1, db1))

        a0, a1, b0, b1 = plsc.parallel_loop(
            0, N // (2 * L), carry=(inf_vec, inf_vec, inf_vec, inf_vec),
            unroll=2)(pa)
        finish_pair(a0, a1, b0, b1, m, (qxa, qya, qza), (qxb, qyb, qzb))
        return 0

    lax.fori_loop(0, ROWS_PER_W // 2, row_body, 0)
    pltpu.sync_copy(outv, out_hbm.at[pl.ds((b * M + row0) * K,
                                           ROWS_PER_W * K)])


@jax.jit
def _knn_sc(x1t, x2t):
    mesh = plsc.VectorSubcoreMesh(core_axis_name="c", subcore_axis_name="s")
    f = functools.partial(
        pl.kernel,
        out_type=jax.ShapeDtypeStruct((B * M * K,), jnp.int32),
        mesh=mesh,
        compiler_params=pltpu.CompilerParams(needs_layout_passes=False),
        scratch_types=[
            pltpu.VMEM((N + L,), jnp.float32),
            pltpu.VMEM((N + L,), jnp.float32),
            pltpu.VMEM((N + L,), jnp.float32),
            pltpu.VMEM((ROWS_PER_W,), jnp.float32),
            pltpu.VMEM((ROWS_PER_W,), jnp.float32),
            pltpu.VMEM((ROWS_PER_W,), jnp.float32),
            pltpu.VMEM((N,), jnp.float32),
            pltpu.VMEM((N + L,), jnp.float32),
            pltpu.VMEM((N + L,), jnp.float32),
            pltpu.VMEM((N + 4 * L,), jnp.int32),
            pltpu.VMEM((N + 4 * L,), jnp.int32),
            pltpu.VMEM((ROWS_PER_W * K,), jnp.int32),
        ],
    )(_knn_body)
    return f(x1t, x2t)


def kernel(xyz1, xyz2):
    x1t = xyz1.transpose(0, 2, 1).reshape(B * 3, N)
    x2t = xyz2.transpose(0, 2, 1).reshape(B * 3, M)
    out = _knn_sc(x1t, x2t)
    return out.reshape(B, M, K, 1)
